# baseline (speedup 1.0000x reference)
import functools

import jax
import jax.numpy as jnp
from jax import lax
from jax.experimental import pallas as pl
from jax.experimental.pallas import tpu as pltpu
from jax.experimental.pallas import tpu_sc as plsc

_NP = 20000
_NPAD = 20480
_CHUNK = 2560
_NCH = _NPAD // _CHUNK
_NOBJ = 64
_THR = 0.6

_NW = 32
_EPW = 8 * _NP * 4 // _NW
_RPW = _EPW // 4
_ITERS = _EPW // 16


def _tc_body(pr_ref, ox1_ref, oy1_ref, ox2_ref, oy2_ref, sc_ref,
             idx_out, sco_out, colmax, colarg, rval, ridx):
    b = pl.program_id(0)
    ph = pl.program_id(1)
    c = pl.program_id(2)
    nb = pl.num_programs(0)

    @pl.when(jnp.logical_and(jnp.logical_and(b == 0, ph == 0), c == 0))
    def _init():
        sco_out[0, 0] = 0.0

    @pl.when(jnp.logical_and(ph == 0, c == 0))
    def _reset():
        rval[...] = jnp.full_like(rval[...], -1.0)
        ridx[...] = jnp.zeros_like(ridx[...])

    glob = c * _CHUNK + jax.lax.broadcasted_iota(jnp.int32, (1, _CHUNK), 1)
    jcol = jax.lax.broadcasted_iota(jnp.int32, (_NOBJ, _CHUNK), 0)

    @pl.when(ph == 0)
    def _phase_a():
        px1 = pr_ref[0:1, :]
        py1 = pr_ref[1:2, :]
        px2 = pr_ref[2:3, :]
        py2 = pr_ref[3:4, :]
        bx1 = ox1_ref[0]
        by1 = oy1_ref[0]
        bx2 = ox2_ref[0]
        by2 = oy2_ref[0]
        iw = jnp.maximum(jnp.minimum(bx2, px2) - jnp.maximum(bx1, px1), 0.0)
        ih = jnp.maximum(jnp.minimum(by2, py2) - jnp.maximum(by1, py1), 0.0)
        inter = iw * ih
        area_o = (bx2 - bx1) * (by2 - by1)
        area_p = (px2 - px1) * (py2 - py1)
        union = jnp.maximum(area_o + area_p - inter, 1e-10)
        iou = inter / union

        cm = jnp.max(iou, axis=0, keepdims=True)
        ca = jnp.min(jnp.where(iou == cm, jcol, _NOBJ), axis=0, keepdims=True)
        colmax[:, pl.ds(c * _CHUNK, _CHUNK)] = cm
        colarg[:, pl.ds(c * _CHUNK, _CHUNK)] = ca

        rm = jnp.max(iou, axis=1, keepdims=True)
        ri = jnp.min(jnp.where(iou == rm, glob, _NPAD), axis=1, keepdims=True)
        upd = rm > rval[...]
        rval[...] = jnp.where(upd, rm, rval[...])
        ridx[...] = jnp.where(upd, ri, ridx[...])

    @pl.when(ph == 1)
    def _phase_b():
        cm = colmax[:, pl.ds(c * _CHUNK, _CHUNK)]
        ca = colarg[:, pl.ds(c * _CHUNK, _CHUNK)]
        pfe = ridx[...]
        match = pfe == glob
        forced = jnp.max(jnp.where(match, 1, 0), axis=0, keepdims=True) > 0
        assigned = jnp.max(jnp.where(match, jcol, -1), axis=0, keepdims=True)
        obj = jnp.where(forced, assigned, ca)
        idx_out[0] = b * _NOBJ + obj

        @pl.when(b == nb - 1)
        def _score():
            valid = glob < _NP
            s0 = sc_ref[0:1, :]
            s1 = sc_ref[1:2, :]
            m = jnp.maximum(s0, s1)
            lse = m + jnp.log(jnp.exp(s0 - m) + jnp.exp(s1 - m))
            lbl = jnp.logical_or(forced, cm >= _THR)
            lp = jnp.where(lbl, s1, s0) - lse
            sco_out[0, 0] += jnp.sum(jnp.where(valid, lp, 0.0))


@functools.partial(
    pl.kernel,
    mesh=plsc.VectorSubcoreMesh(core_axis_name="c", subcore_axis_name="s"),
    out_type=jax.ShapeDtypeStruct((_NW, 16), jnp.float32),
    compiler_params=pltpu.CompilerParams(
        needs_layout_passes=False, use_tc_tiling_on_sc=False),
    scratch_types=[
        pltpu.VMEM((_RPW,), jnp.int32),
        pltpu.VMEM((_RPW, 4), jnp.float32),
        pltpu.VMEM((8, _NOBJ, 4), jnp.float32),
        pltpu.VMEM((16,), jnp.float32),
    ],
)
def _sc_loc(idx_hbm, pred_hbm, tab_hbm, out_hbm, idx_v, pred_v, tab_v, acc_v):
    wid = lax.axis_index("s") * 2 + lax.axis_index("c")
    b = wid >> 2
    off = (wid & 3) * _RPW
    pltpu.sync_copy(
        idx_hbm.at[b, 0, pl.ds(pl.multiple_of(off, 8), _RPW)], idx_v)
    pltpu.sync_copy(
        pred_hbm.at[b, pl.ds(pl.multiple_of(off, 8), _RPW)], pred_v)
    pltpu.sync_copy(tab_hbm, tab_v)
    lane = lax.iota(jnp.int32, 16)
    rowsel = lane >> 2
    coord = lane & 3

    def body(i, tot):
        row = i * 4 + rowsel
        flatobj = plsc.load_gather(idx_v, [row])
        t = plsc.load_gather(tab_v, [flatobj >> 6, flatobj & 63, coord])
        p = plsc.load_gather(pred_v, [row, coord])
        return tot + jnp.abs(p - t)

    acc_v[...] = lax.fori_loop(0, _ITERS, body, jnp.zeros((16,), jnp.float32))
    pltpu.sync_copy(acc_v, out_hbm.at[wid])


def kernel(predicted_boxes, predicted_scores, boxes, prior_boxes):
    bsz = predicted_boxes.shape[0]
    pad = _NPAD - _NP
    prT = jnp.pad(prior_boxes, ((0, pad), (0, 0))).T
    ox1 = boxes[..., 0:1]
    oy1 = boxes[..., 1:2]
    ox2 = boxes[..., 2:3]
    oy2 = boxes[..., 3:4]
    scT = jnp.pad(predicted_scores, ((0, pad), (0, 0))).T

    idx, sco_sum = pl.pallas_call(
        _tc_body,
        grid=(bsz, 2, _NCH),
        in_specs=[
            pl.BlockSpec((4, _CHUNK), lambda b, ph, c: (0, c)),
            pl.BlockSpec((1, _NOBJ, 1), lambda b, ph, c: (b, 0, 0)),
            pl.BlockSpec((1, _NOBJ, 1), lambda b, ph, c: (b, 0, 0)),
            pl.BlockSpec((1, _NOBJ, 1), lambda b, ph, c: (b, 0, 0)),
            pl.BlockSpec((1, _NOBJ, 1), lambda b, ph, c: (b, 0, 0)),
            pl.BlockSpec((2, _CHUNK), lambda b, ph, c: (0, c)),
        ],
        out_specs=[
            pl.BlockSpec((1, 1, _CHUNK), lambda b, ph, c: (b, 0, c)),
            pl.BlockSpec((1, 1), lambda b, ph, c: (0, 0),
                         memory_space=pltpu.SMEM),
        ],
        out_shape=[
            jax.ShapeDtypeStruct((bsz, 1, _NPAD), jnp.int32),
            jax.ShapeDtypeStruct((1, 1), jnp.float32),
        ],
        scratch_shapes=[
            pltpu.VMEM((1, _NPAD), jnp.float32),
            pltpu.VMEM((1, _NPAD), jnp.int32),
            pltpu.VMEM((_NOBJ, 1), jnp.float32),
            pltpu.VMEM((_NOBJ, 1), jnp.int32),
        ],
    )(prT, ox1, oy1, ox2, oy2, scT)

    loc_parts = _sc_loc(idx, predicted_boxes, boxes)

    loc_loss = jnp.sum(loc_parts) / (bsz * _NP * 4)
    score_loss = -sco_sum[0, 0] / _NP
    return score_loss + loc_loss


# speedup vs baseline: 1.2569x; 1.2569x over previous
import functools

import jax
import jax.numpy as jnp
from jax import lax
from jax.experimental import pallas as pl
from jax.experimental.pallas import tpu as pltpu
from jax.experimental.pallas import tpu_sc as plsc

_NP = 20000
_NPAD = 20480
_CHUNK = 2560
_NCH = _NPAD // _CHUNK
_NOBJ = 64
_THR = 0.6

_NW = 32
_EPW = 8 * _NP * 4 // _NW
_RPW = _EPW // 4
_ITERS = _EPW // 16


def _tc_body(pr_ref, ox1_ref, oy1_ref, ox2_ref, oy2_ref, sc_ref,
             idx_out, sco_out, colmax, colarg, rval, ridx):
    b = pl.program_id(0)
    ph = pl.program_id(1)
    c = pl.program_id(2)
    nb = pl.num_programs(0)

    @pl.when(jnp.logical_and(jnp.logical_and(b == 0, ph == 0), c == 0))
    def _init():
        sco_out[0, 0] = 0.0

    @pl.when(jnp.logical_and(ph == 0, c == 0))
    def _reset():
        rval[...] = jnp.full_like(rval[...], -1.0)
        ridx[...] = jnp.zeros_like(ridx[...])

    glob = c * _CHUNK + jax.lax.broadcasted_iota(jnp.int32, (1, _CHUNK), 1)
    jcol = jax.lax.broadcasted_iota(jnp.int32, (_NOBJ, _CHUNK), 0)

    @pl.when(ph == 0)
    def _phase_a():
        px1 = pr_ref[0:1, :]
        py1 = pr_ref[1:2, :]
        px2 = pr_ref[2:3, :]
        py2 = pr_ref[3:4, :]
        bx1 = ox1_ref[0]
        by1 = oy1_ref[0]
        bx2 = ox2_ref[0]
        by2 = oy2_ref[0]
        iw = jnp.maximum(jnp.minimum(bx2, px2) - jnp.maximum(bx1, px1), 0.0)
        ih = jnp.maximum(jnp.minimum(by2, py2) - jnp.maximum(by1, py1), 0.0)
        inter = iw * ih
        area_o = (bx2 - bx1) * (by2 - by1)
        area_p = (px2 - px1) * (py2 - py1)
        union = jnp.maximum(area_o + area_p - inter, 1e-10)
        iou = inter / union

        cm = jnp.max(iou, axis=0, keepdims=True)
        ca = jnp.min(jnp.where(iou == cm, jcol, _NOBJ), axis=0, keepdims=True)
        colmax[:, pl.ds(c * _CHUNK, _CHUNK)] = cm
        colarg[:, pl.ds(c * _CHUNK, _CHUNK)] = ca

        rm = jnp.max(iou, axis=1, keepdims=True)
        ri = jnp.min(jnp.where(iou == rm, glob, _NPAD), axis=1, keepdims=True)
        upd = rm > rval[...]
        rval[...] = jnp.where(upd, rm, rval[...])
        ridx[...] = jnp.where(upd, ri, ridx[...])

    @pl.when(ph == 1)
    def _phase_b():
        cm = colmax[:, pl.ds(c * _CHUNK, _CHUNK)]
        ca = colarg[:, pl.ds(c * _CHUNK, _CHUNK)]
        pfe = ridx[...]
        match = pfe == glob
        forced = jnp.max(jnp.where(match, 1, 0), axis=0, keepdims=True) > 0
        assigned = jnp.max(jnp.where(match, jcol, -1), axis=0, keepdims=True)
        obj = jnp.where(forced, assigned, ca)
        idx_out[0] = b * _NOBJ + obj

        @pl.when(b == nb - 1)
        def _score():
            valid = glob < _NP
            s0 = sc_ref[0:1, :]
            s1 = sc_ref[1:2, :]
            m = jnp.maximum(s0, s1)
            lse = m + jnp.log(jnp.exp(s0 - m) + jnp.exp(s1 - m))
            lbl = jnp.logical_or(forced, cm >= _THR)
            lp = jnp.where(lbl, s1, s0) - lse
            sco_out[0, 0] += jnp.sum(jnp.where(valid, lp, 0.0))


@functools.partial(
    pl.kernel,
    mesh=plsc.VectorSubcoreMesh(core_axis_name="c", subcore_axis_name="s"),
    out_type=jax.ShapeDtypeStruct((_NW, 16), jnp.float32),
    compiler_params=pltpu.CompilerParams(needs_layout_passes=False),
    scratch_types=[
        pltpu.VMEM((5120,), jnp.int32),
        pltpu.VMEM((_EPW,), jnp.float32),
        pltpu.VMEM((512 * 4,), jnp.float32),
        pltpu.VMEM((16,), jnp.float32),
    ],
)
def _sc_loc(idx_hbm, pred_hbm, tab_hbm, out_hbm, idx_v, pred_v, tab_v, acc_v):
    wid = lax.axis_index("s") * 2 + lax.axis_index("c")
    b = wid >> 2
    off = (wid & 3) * _RPW
    aligned = (off >> 7) << 7
    shift = off - aligned
    pltpu.sync_copy(
        idx_hbm.at[b, 0, pl.ds(pl.multiple_of(aligned, 128), 5120)], idx_v)
    pltpu.sync_copy(pred_hbm.at[wid], pred_v)
    pltpu.sync_copy(tab_hbm, tab_v)
    lane = lax.iota(jnp.int32, 16)
    rowsel = lane >> 2
    coord = lane & 3

    def body(i, tot):
        flatobj = plsc.load_gather(idx_v, [shift + i * 4 + rowsel])
        t = plsc.load_gather(tab_v, [flatobj * 4 + coord])
        p = pred_v[pl.ds(i * 16, 16)]
        return tot + jnp.abs(p - t)

    acc_v[...] = lax.fori_loop(0, _ITERS, body, jnp.zeros((16,), jnp.float32))
    pltpu.sync_copy(acc_v, out_hbm.at[wid])


def kernel(predicted_boxes, predicted_scores, boxes, prior_boxes):
    bsz = predicted_boxes.shape[0]
    pad = _NPAD - _NP
    prT = jnp.pad(prior_boxes, ((0, pad), (0, 0))).T
    ox1 = boxes[..., 0:1]
    oy1 = boxes[..., 1:2]
    ox2 = boxes[..., 2:3]
    oy2 = boxes[..., 3:4]
    scT = jnp.pad(predicted_scores, ((0, pad), (0, 0))).T

    idx, sco_sum = pl.pallas_call(
        _tc_body,
        grid=(bsz, 2, _NCH),
        in_specs=[
            pl.BlockSpec((4, _CHUNK), lambda b, ph, c: (0, c)),
            pl.BlockSpec((1, _NOBJ, 1), lambda b, ph, c: (b, 0, 0)),
            pl.BlockSpec((1, _NOBJ, 1), lambda b, ph, c: (b, 0, 0)),
            pl.BlockSpec((1, _NOBJ, 1), lambda b, ph, c: (b, 0, 0)),
            pl.BlockSpec((1, _NOBJ, 1), lambda b, ph, c: (b, 0, 0)),
            pl.BlockSpec((2, _CHUNK), lambda b, ph, c: (0, c)),
        ],
        out_specs=[
            pl.BlockSpec((1, 1, _CHUNK), lambda b, ph, c: (b, 0, c)),
            pl.BlockSpec((1, 1), lambda b, ph, c: (0, 0),
                         memory_space=pltpu.SMEM),
        ],
        out_shape=[
            jax.ShapeDtypeStruct((bsz, 1, _NPAD), jnp.int32),
            jax.ShapeDtypeStruct((1, 1), jnp.float32),
        ],
        scratch_shapes=[
            pltpu.VMEM((1, _NPAD), jnp.float32),
            pltpu.VMEM((1, _NPAD), jnp.int32),
            pltpu.VMEM((_NOBJ, 1), jnp.float32),
            pltpu.VMEM((_NOBJ, 1), jnp.int32),
        ],
    )(prT, ox1, oy1, ox2, oy2, scT)

    pred_w = predicted_boxes.reshape(_NW, _EPW)
    tab = boxes.reshape(bsz * _NOBJ * 4)
    loc_parts = _sc_loc(idx, pred_w, tab)

    loc_loss = jnp.sum(loc_parts) / (bsz * _NP * 4)
    score_loss = -sco_sum[0, 0] / _NP
    return score_loss + loc_loss


# speedup vs baseline: 1.8628x; 1.4821x over previous
import functools

import jax
import jax.numpy as jnp
from jax import lax
from jax.experimental import pallas as pl
from jax.experimental.pallas import tpu as pltpu
from jax.experimental.pallas import tpu_sc as plsc

_NP = 20000
_NPAD = 20480
_CHUNK = 2560
_NCH = _NPAD // _CHUNK
_NOBJ = 64
_THR = 0.6
_NW = 32
_PPW = 16


def _tc_body(pr_ref, ox1_ref, oy1_ref, ox2_ref, oy2_ref, bt_ref, pred_ref,
             sc_ref, pp_out, ca0t_out, win_out, loc_out, sco_out,
             colmax, colarg, rval, ridx, aca0, aasg, ap0, ap1, ap2, ap3):
    b = pl.program_id(0)
    ph = pl.program_id(1)
    c = pl.program_id(2)
    nb = pl.num_programs(0)

    @pl.when(jnp.logical_and(jnp.logical_and(b == 0, ph == 0), c == 0))
    def _init():
        loc_out[0, 0] = 0.0
        sco_out[0, 0] = 0.0

    @pl.when(jnp.logical_and(ph == 0, c == 0))
    def _reset():
        rval[...] = jnp.full_like(rval[...], -1.0)
        ridx[...] = jnp.zeros_like(ridx[...])

    @pl.when(jnp.logical_and(ph == 1, c == 0))
    def _reset_b():
        aca0[...] = jnp.zeros_like(aca0[...])
        aasg[...] = jnp.zeros_like(aasg[...])
        ap0[...] = jnp.zeros_like(ap0[...])
        ap1[...] = jnp.zeros_like(ap1[...])
        ap2[...] = jnp.zeros_like(ap2[...])
        ap3[...] = jnp.zeros_like(ap3[...])

    glob = c * _CHUNK + jax.lax.broadcasted_iota(jnp.int32, (1, _CHUNK), 1)
    jcol = jax.lax.broadcasted_iota(jnp.int32, (_NOBJ, _CHUNK), 0)

    @pl.when(ph == 0)
    def _phase_a():
        px1 = pr_ref[0:1, :]
        py1 = pr_ref[1:2, :]
        px2 = pr_ref[2:3, :]
        py2 = pr_ref[3:4, :]
        bx1 = ox1_ref[0]
        by1 = oy1_ref[0]
        bx2 = ox2_ref[0]
        by2 = oy2_ref[0]
        iw = jnp.maximum(jnp.minimum(bx2, px2) - jnp.maximum(bx1, px1), 0.0)
        ih = jnp.maximum(jnp.minimum(by2, py2) - jnp.maximum(by1, py1), 0.0)
        inter = iw * ih
        area_o = (bx2 - bx1) * (by2 - by1)
        area_p = (px2 - px1) * (py2 - py1)
        union = jnp.maximum(area_o + area_p - inter, 1e-10)
        iou = inter / union

        cm = jnp.max(iou, axis=0, keepdims=True)
        ca = jnp.min(jnp.where(iou == cm, jcol, _NOBJ), axis=0, keepdims=True)
        colmax[:, pl.ds(c * _CHUNK, _CHUNK)] = cm
        colarg[:, pl.ds(c * _CHUNK, _CHUNK)] = ca

        rm = jnp.max(iou, axis=1, keepdims=True)
        ri = jnp.min(jnp.where(iou == rm, glob, _NPAD), axis=1, keepdims=True)
        upd = rm > rval[...]
        rval[...] = jnp.where(upd, rm, rval[...])
        ridx[...] = jnp.where(upd, ri, ridx[...])

    @pl.when(ph == 1)
    def _phase_b():
        cm = colmax[:, pl.ds(c * _CHUNK, _CHUNK)]
        ca = colarg[:, pl.ds(c * _CHUNK, _CHUNK)]
        pfe = ridx[...]
        match = pfe == glob
        forced = jnp.max(jnp.where(match, 1, 0), axis=0, keepdims=True) > 0
        assigned = jnp.max(jnp.where(match, jcol, -1), axis=0, keepdims=True)

        caf = ca.astype(jnp.float32)
        asgf = assigned.astype(jnp.float32)
        aca0[...] += jnp.sum(jnp.where(match, caf, 0.0), axis=1, keepdims=True)
        aasg[...] += jnp.sum(jnp.where(match, asgf, 0.0), axis=1, keepdims=True)

        oh = (jcol == ca).astype(jnp.float32)
        bt = bt_ref[0]
        tl = jax.lax.dot_general(bt, oh, (((1,), (0,)), ((), ())),
                                 preferred_element_type=jnp.float32)
        pred = pred_ref[0]
        valid = glob < _NP
        loc_out[0, 0] += jnp.sum(jnp.where(valid, jnp.abs(pred - tl), 0.0))

        ap0[...] += jnp.sum(jnp.where(match, pred[0:1, :], 0.0), axis=1,
                            keepdims=True)
        ap1[...] += jnp.sum(jnp.where(match, pred[1:2, :], 0.0), axis=1,
                            keepdims=True)
        ap2[...] += jnp.sum(jnp.where(match, pred[2:3, :], 0.0), axis=1,
                            keepdims=True)
        ap3[...] += jnp.sum(jnp.where(match, pred[3:4, :], 0.0), axis=1,
                            keepdims=True)

        @pl.when(c == _NCH - 1)
        def _emit_meta():
            jrow = jax.lax.broadcasted_iota(jnp.int32, (_NOBJ, 1), 0)
            pp_out[...] = jnp.concatenate(
                [ap0[...], ap1[...], ap2[...], ap3[...]], axis=1)
            ca0t_out[...] = b * _NOBJ + aca0[...].astype(jnp.int32)
            win_out[...] = (aasg[...].astype(jnp.int32) == jrow).astype(
                jnp.float32)

        @pl.when(b == nb - 1)
        def _score():
            s0 = sc_ref[0:1, :]
            s1 = sc_ref[1:2, :]
            m = jnp.maximum(s0, s1)
            lse = m + jnp.log(jnp.exp(s0 - m) + jnp.exp(s1 - m))
            lbl = jnp.logical_or(forced, cm >= _THR)
            lp = jnp.where(lbl, s1, s0) - lse
            sco_out[0, 0] += jnp.sum(jnp.where(valid, lp, 0.0))


@functools.partial(
    pl.kernel,
    mesh=plsc.VectorSubcoreMesh(core_axis_name="c", subcore_axis_name="s"),
    out_type=jax.ShapeDtypeStruct((_NW * 16,), jnp.float32),
    compiler_params=pltpu.CompilerParams(needs_layout_passes=False),
    scratch_types=[
        pltpu.VMEM((_PPW, 1), jnp.int32),
        pltpu.VMEM((_PPW, 1), jnp.float32),
        pltpu.VMEM((_PPW, 4), jnp.float32),
        pltpu.VMEM((8 * _NOBJ * 4,), jnp.float32),
        pltpu.VMEM((16,), jnp.float32),
    ],
)
def _sc_corr(pp_hbm, ca0t_hbm, win_hbm, tab_hbm, out_hbm,
             ca0_v, win_v, rows_v, tab_v, acc_v):
    wid = lax.axis_index("s") * 2 + lax.axis_index("c")
    b = wid >> 2
    q = (wid & 3) * _PPW
    woff = pl.multiple_of(wid * _PPW, 8)
    pltpu.sync_copy(pp_hbm.at[pl.ds(woff, _PPW)], rows_v)
    pltpu.sync_copy(ca0t_hbm.at[pl.ds(woff, _PPW)], ca0_v)
    pltpu.sync_copy(win_hbm.at[pl.ds(woff, _PPW)], win_v)
    pltpu.sync_copy(tab_hbm, tab_v)
    lane = lax.iota(jnp.int32, 16)
    lz = lane * 0
    asg_t = (b * _NOBJ + q + lane) * 4
    ca0_i = plsc.load_gather(ca0_v, [lane, lz])
    win = plsc.load_gather(win_v, [lane, lz])
    corr = jnp.zeros((16,), jnp.float32)
    for cc in range(4):
        p = plsc.load_gather(rows_v, [lane, lz + cc])
        ta = plsc.load_gather(tab_v, [asg_t + cc])
        tc0 = plsc.load_gather(tab_v, [ca0_i * 4 + cc])
        corr = corr + win * (jnp.abs(p - ta) - jnp.abs(p - tc0))
    acc_v[...] = corr
    pltpu.sync_copy(acc_v, out_hbm.at[pl.ds(pl.multiple_of(wid * 16, 8), 16)])


def kernel(predicted_boxes, predicted_scores, boxes, prior_boxes):
    bsz = predicted_boxes.shape[0]
    pad = _NPAD - _NP
    prT = jnp.pad(prior_boxes, ((0, pad), (0, 0))).T
    ox1 = boxes[..., 0:1]
    oy1 = boxes[..., 1:2]
    ox2 = boxes[..., 2:3]
    oy2 = boxes[..., 3:4]
    bT = jnp.swapaxes(boxes, 1, 2)
    predT = jnp.swapaxes(
        jnp.pad(predicted_boxes, ((0, 0), (0, pad), (0, 0))), 1, 2)
    scT = jnp.pad(predicted_scores, ((0, pad), (0, 0))).T

    pp, ca0t, win, loc_sum, sco_sum = pl.pallas_call(
        _tc_body,
        grid=(bsz, 2, _NCH),
        in_specs=[
            pl.BlockSpec((4, _CHUNK), lambda b, ph, c: (0, c)),
            pl.BlockSpec((1, _NOBJ, 1), lambda b, ph, c: (b, 0, 0)),
            pl.BlockSpec((1, _NOBJ, 1), lambda b, ph, c: (b, 0, 0)),
            pl.BlockSpec((1, _NOBJ, 1), lambda b, ph, c: (b, 0, 0)),
            pl.BlockSpec((1, _NOBJ, 1), lambda b, ph, c: (b, 0, 0)),
            pl.BlockSpec((1, 4, _NOBJ), lambda b, ph, c: (b, 0, 0)),
            pl.BlockSpec((1, 4, _CHUNK), lambda b, ph, c: (b, 0, c)),
            pl.BlockSpec((2, _CHUNK), lambda b, ph, c: (0, c)),
        ],
        out_specs=[
            pl.BlockSpec((_NOBJ, 4), lambda b, ph, c: (b, 0)),
            pl.BlockSpec((_NOBJ, 1), lambda b, ph, c: (b, 0)),
            pl.BlockSpec((_NOBJ, 1), lambda b, ph, c: (b, 0)),
            pl.BlockSpec((1, 1), lambda b, ph, c: (0, 0),
                         memory_space=pltpu.SMEM),
            pl.BlockSpec((1, 1), lambda b, ph, c: (0, 0),
                         memory_space=pltpu.SMEM),
        ],
        out_shape=[
            jax.ShapeDtypeStruct((bsz * _NOBJ, 4), jnp.float32),
            jax.ShapeDtypeStruct((bsz * _NOBJ, 1), jnp.int32),
            jax.ShapeDtypeStruct((bsz * _NOBJ, 1), jnp.float32),
            jax.ShapeDtypeStruct((1, 1), jnp.float32),
            jax.ShapeDtypeStruct((1, 1), jnp.float32),
        ],
        scratch_shapes=[
            pltpu.VMEM((1, _NPAD), jnp.float32),
            pltpu.VMEM((1, _NPAD), jnp.int32),
            pltpu.VMEM((_NOBJ, 1), jnp.float32),
            pltpu.VMEM((_NOBJ, 1), jnp.int32),
            pltpu.VMEM((_NOBJ, 1), jnp.float32),
            pltpu.VMEM((_NOBJ, 1), jnp.float32),
            pltpu.VMEM((_NOBJ, 1), jnp.float32),
            pltpu.VMEM((_NOBJ, 1), jnp.float32),
            pltpu.VMEM((_NOBJ, 1), jnp.float32),
            pltpu.VMEM((_NOBJ, 1), jnp.float32),
        ],
    )(prT, ox1, oy1, ox2, oy2, bT, predT, scT)

    tab = boxes.reshape(bsz * _NOBJ * 4)
    corr_parts = _sc_corr(pp, ca0t, win, tab)

    loc_loss = (loc_sum[0, 0] + jnp.sum(corr_parts)) / (bsz * _NP * 4)
    score_loss = -sco_sum[0, 0] / _NP
    return score_loss + loc_loss


# speedup vs baseline: 2.1381x; 1.1478x over previous
import functools

import jax
import jax.numpy as jnp
from jax import lax
from jax.experimental import pallas as pl
from jax.experimental.pallas import tpu as pltpu
from jax.experimental.pallas import tpu_sc as plsc

_NP = 20000
_NPAD = 20480
_CHUNK = 5120
_NCH = _NPAD // _CHUNK
_NOBJ = 64
_THR = 0.6
_NW = 32
_PPW = 16


def _tc_body(pr_ref, ox1_ref, oy1_ref, ox2_ref, oy2_ref, bt_ref, pred_ref,
             sc_ref, pp_out, ca0t_out, win_out, loc_out, sco_out,
             colmax, colarg, rval, ridx, aca0, aasg, ap0, ap1, ap2, ap3):
    b = pl.program_id(0)
    ph = pl.program_id(1)
    c = pl.program_id(2)
    nb = pl.num_programs(0)

    @pl.when(jnp.logical_and(jnp.logical_and(b == 0, ph == 0), c == 0))
    def _init():
        loc_out[0, 0] = 0.0
        sco_out[0, 0] = 0.0

    @pl.when(jnp.logical_and(ph == 0, c == 0))
    def _reset():
        rval[...] = jnp.full_like(rval[...], -1.0)
        ridx[...] = jnp.zeros_like(ridx[...])

    @pl.when(jnp.logical_and(ph == 1, c == 0))
    def _reset_b():
        aca0[...] = jnp.zeros_like(aca0[...])
        aasg[...] = jnp.zeros_like(aasg[...])
        ap0[...] = jnp.zeros_like(ap0[...])
        ap1[...] = jnp.zeros_like(ap1[...])
        ap2[...] = jnp.zeros_like(ap2[...])
        ap3[...] = jnp.zeros_like(ap3[...])

    glob = c * _CHUNK + jax.lax.broadcasted_iota(jnp.int32, (1, _CHUNK), 1)
    jcol = jax.lax.broadcasted_iota(jnp.int32, (_NOBJ, _CHUNK), 0)

    @pl.when(ph == 0)
    def _phase_a():
        px1 = pr_ref[0:1, :]
        py1 = pr_ref[1:2, :]
        px2 = pr_ref[2:3, :]
        py2 = pr_ref[3:4, :]
        bx1 = ox1_ref[0]
        by1 = oy1_ref[0]
        bx2 = ox2_ref[0]
        by2 = oy2_ref[0]
        iw = jnp.maximum(jnp.minimum(bx2, px2) - jnp.maximum(bx1, px1), 0.0)
        ih = jnp.maximum(jnp.minimum(by2, py2) - jnp.maximum(by1, py1), 0.0)
        inter = iw * ih
        area_o = (bx2 - bx1) * (by2 - by1)
        area_p = (px2 - px1) * (py2 - py1)
        union = jnp.maximum(area_o + area_p - inter, 1e-10)
        iou = inter / union

        cm = jnp.max(iou, axis=0, keepdims=True)
        ca = jnp.min(jnp.where(iou == cm, jcol, _NOBJ), axis=0, keepdims=True)
        colmax[:, pl.ds(c * _CHUNK, _CHUNK)] = cm
        colarg[:, pl.ds(c * _CHUNK, _CHUNK)] = ca

        rm = jnp.max(iou, axis=1, keepdims=True)
        ri = jnp.min(jnp.where(iou == rm, glob, _NPAD), axis=1, keepdims=True)
        upd = rm > rval[...]
        rval[...] = jnp.where(upd, rm, rval[...])
        ridx[...] = jnp.where(upd, ri, ridx[...])

    @pl.when(ph == 1)
    def _phase_b():
        cm = colmax[:, pl.ds(c * _CHUNK, _CHUNK)]
        ca = colarg[:, pl.ds(c * _CHUNK, _CHUNK)]
        pfe = ridx[...]
        match = pfe == glob
        forced = jnp.max(jnp.where(match, 1, 0), axis=0, keepdims=True) > 0
        assigned = jnp.max(jnp.where(match, jcol, -1), axis=0, keepdims=True)

        caf = ca.astype(jnp.float32)
        asgf = assigned.astype(jnp.float32)
        aca0[...] += jnp.sum(jnp.where(match, caf, 0.0), axis=1, keepdims=True)
        aasg[...] += jnp.sum(jnp.where(match, asgf, 0.0), axis=1, keepdims=True)

        oh = (jcol == ca).astype(jnp.float32)
        bt = bt_ref[0]
        tl = jax.lax.dot_general(bt, oh, (((1,), (0,)), ((), ())),
                                 preferred_element_type=jnp.float32)
        pred = pred_ref[0]
        valid = glob < _NP
        loc_out[0, 0] += jnp.sum(jnp.where(valid, jnp.abs(pred - tl), 0.0))

        ap0[...] += jnp.sum(jnp.where(match, pred[0:1, :], 0.0), axis=1,
                            keepdims=True)
        ap1[...] += jnp.sum(jnp.where(match, pred[1:2, :], 0.0), axis=1,
                            keepdims=True)
        ap2[...] += jnp.sum(jnp.where(match, pred[2:3, :], 0.0), axis=1,
                            keepdims=True)
        ap3[...] += jnp.sum(jnp.where(match, pred[3:4, :], 0.0), axis=1,
                            keepdims=True)

        @pl.when(c == _NCH - 1)
        def _emit_meta():
            jrow = jax.lax.broadcasted_iota(jnp.int32, (_NOBJ, 1), 0)
            pp_out[...] = jnp.concatenate(
                [ap0[...], ap1[...], ap2[...], ap3[...]], axis=1)
            ca0t_out[...] = b * _NOBJ + aca0[...].astype(jnp.int32)
            win_out[...] = (aasg[...].astype(jnp.int32) == jrow).astype(
                jnp.float32)

        @pl.when(b == nb - 1)
        def _score():
            s0 = sc_ref[0:1, :]
            s1 = sc_ref[1:2, :]
            m = jnp.maximum(s0, s1)
            lse = m + jnp.log(jnp.exp(s0 - m) + jnp.exp(s1 - m))
            lbl = jnp.logical_or(forced, cm >= _THR)
            lp = jnp.where(lbl, s1, s0) - lse
            sco_out[0, 0] += jnp.sum(jnp.where(valid, lp, 0.0))


@functools.partial(
    pl.kernel,
    mesh=plsc.VectorSubcoreMesh(core_axis_name="c", subcore_axis_name="s"),
    out_type=jax.ShapeDtypeStruct((_NW * 16,), jnp.float32),
    compiler_params=pltpu.CompilerParams(needs_layout_passes=False),
    scratch_types=[
        pltpu.VMEM((_PPW, 1), jnp.int32),
        pltpu.VMEM((_PPW, 1), jnp.float32),
        pltpu.VMEM((_PPW, 4), jnp.float32),
        pltpu.VMEM((8 * _NOBJ * 4,), jnp.float32),
        pltpu.VMEM((16,), jnp.float32),
    ],
)
def _sc_corr(pp_hbm, ca0t_hbm, win_hbm, tab_hbm, out_hbm,
             ca0_v, win_v, rows_v, tab_v, acc_v):
    wid = lax.axis_index("s") * 2 + lax.axis_index("c")
    b = wid >> 2
    q = (wid & 3) * _PPW
    woff = pl.multiple_of(wid * _PPW, 8)
    pltpu.sync_copy(pp_hbm.at[pl.ds(woff, _PPW)], rows_v)
    pltpu.sync_copy(ca0t_hbm.at[pl.ds(woff, _PPW)], ca0_v)
    pltpu.sync_copy(win_hbm.at[pl.ds(woff, _PPW)], win_v)
    pltpu.sync_copy(tab_hbm, tab_v)
    lane = lax.iota(jnp.int32, 16)
    lz = lane * 0
    asg_t = (b * _NOBJ + q + lane) * 4
    ca0_i = plsc.load_gather(ca0_v, [lane, lz])
    win = plsc.load_gather(win_v, [lane, lz])
    corr = jnp.zeros((16,), jnp.float32)
    for cc in range(4):
        p = plsc.load_gather(rows_v, [lane, lz + cc])
        ta = plsc.load_gather(tab_v, [asg_t + cc])
        tc0 = plsc.load_gather(tab_v, [ca0_i * 4 + cc])
        corr = corr + win * (jnp.abs(p - ta) - jnp.abs(p - tc0))
    acc_v[...] = corr
    pltpu.sync_copy(acc_v, out_hbm.at[pl.ds(pl.multiple_of(wid * 16, 8), 16)])


def kernel(predicted_boxes, predicted_scores, boxes, prior_boxes):
    bsz = predicted_boxes.shape[0]
    pad = _NPAD - _NP
    prT = jnp.pad(prior_boxes, ((0, pad), (0, 0))).T
    ox1 = boxes[..., 0:1]
    oy1 = boxes[..., 1:2]
    ox2 = boxes[..., 2:3]
    oy2 = boxes[..., 3:4]
    bT = jnp.swapaxes(boxes, 1, 2)
    predT = jnp.swapaxes(
        jnp.pad(predicted_boxes, ((0, 0), (0, pad), (0, 0))), 1, 2)
    scT = jnp.pad(predicted_scores, ((0, pad), (0, 0))).T

    pp, ca0t, win, loc_sum, sco_sum = pl.pallas_call(
        _tc_body,
        grid=(bsz, 2, _NCH),
        in_specs=[
            pl.BlockSpec((4, _CHUNK), lambda b, ph, c: (0, c)),
            pl.BlockSpec((1, _NOBJ, 1), lambda b, ph, c: (b, 0, 0)),
            pl.BlockSpec((1, _NOBJ, 1), lambda b, ph, c: (b, 0, 0)),
            pl.BlockSpec((1, _NOBJ, 1), lambda b, ph, c: (b, 0, 0)),
            pl.BlockSpec((1, _NOBJ, 1), lambda b, ph, c: (b, 0, 0)),
            pl.BlockSpec((1, 4, _NOBJ), lambda b, ph, c: (b, 0, 0)),
            pl.BlockSpec((1, 4, _CHUNK), lambda b, ph, c: (b, 0, c)),
            pl.BlockSpec((2, _CHUNK), lambda b, ph, c: (0, c)),
        ],
        out_specs=[
            pl.BlockSpec((_NOBJ, 4), lambda b, ph, c: (b, 0)),
            pl.BlockSpec((_NOBJ, 1), lambda b, ph, c: (b, 0)),
            pl.BlockSpec((_NOBJ, 1), lambda b, ph, c: (b, 0)),
            pl.BlockSpec((1, 1), lambda b, ph, c: (0, 0),
                         memory_space=pltpu.SMEM),
            pl.BlockSpec((1, 1), lambda b, ph, c: (0, 0),
                         memory_space=pltpu.SMEM),
        ],
        out_shape=[
            jax.ShapeDtypeStruct((bsz * _NOBJ, 4), jnp.float32),
            jax.ShapeDtypeStruct((bsz * _NOBJ, 1), jnp.int32),
            jax.ShapeDtypeStruct((bsz * _NOBJ, 1), jnp.float32),
            jax.ShapeDtypeStruct((1, 1), jnp.float32),
            jax.ShapeDtypeStruct((1, 1), jnp.float32),
        ],
        scratch_shapes=[
            pltpu.VMEM((1, _NPAD), jnp.float32),
            pltpu.VMEM((1, _NPAD), jnp.int32),
            pltpu.VMEM((_NOBJ, 1), jnp.float32),
            pltpu.VMEM((_NOBJ, 1), jnp.int32),
            pltpu.VMEM((_NOBJ, 1), jnp.float32),
            pltpu.VMEM((_NOBJ, 1), jnp.float32),
            pltpu.VMEM((_NOBJ, 1), jnp.float32),
            pltpu.VMEM((_NOBJ, 1), jnp.float32),
            pltpu.VMEM((_NOBJ, 1), jnp.float32),
            pltpu.VMEM((_NOBJ, 1), jnp.float32),
        ],
    )(prT, ox1, oy1, ox2, oy2, bT, predT, scT)

    tab = boxes.reshape(bsz * _NOBJ * 4)
    corr_parts = _sc_corr(pp, ca0t, win, tab)

    loc_loss = (loc_sum[0, 0] + jnp.sum(corr_parts)) / (bsz * _NP * 4)
    score_loss = -sco_sum[0, 0] / _NP
    return score_loss + loc_loss


# speedup vs baseline: 2.1889x; 1.0238x over previous
import functools

import jax
import jax.numpy as jnp
from jax import lax
from jax.experimental import pallas as pl
from jax.experimental.pallas import tpu as pltpu
from jax.experimental.pallas import tpu_sc as plsc

_NP = 20000
_NPAD = 20480
_CHUNK = 10240
_NCH = _NPAD // _CHUNK
_NOBJ = 64
_THR = 0.6
_NW = 32
_PPW = 16


def _tc_body(pr_ref, ox1_ref, oy1_ref, ox2_ref, oy2_ref, bt_ref, pred_ref,
             sc_ref, pp_out, ca0t_out, win_out, loc_out, sco_out,
             colmax, colarg, rval, ridx, aca0, aasg, ap0, ap1, ap2, ap3):
    b = pl.program_id(0)
    ph = pl.program_id(1)
    c = pl.program_id(2)
    nb = pl.num_programs(0)

    @pl.when(jnp.logical_and(jnp.logical_and(b == 0, ph == 0), c == 0))
    def _init():
        loc_out[0, 0] = 0.0
        sco_out[0, 0] = 0.0

    @pl.when(jnp.logical_and(ph == 0, c == 0))
    def _reset():
        rval[...] = jnp.full_like(rval[...], -1.0)
        ridx[...] = jnp.zeros_like(ridx[...])

    @pl.when(jnp.logical_and(ph == 1, c == 0))
    def _reset_b():
        aca0[...] = jnp.zeros_like(aca0[...])
        aasg[...] = jnp.zeros_like(aasg[...])
        ap0[...] = jnp.zeros_like(ap0[...])
        ap1[...] = jnp.zeros_like(ap1[...])
        ap2[...] = jnp.zeros_like(ap2[...])
        ap3[...] = jnp.zeros_like(ap3[...])

    glob = c * _CHUNK + jax.lax.broadcasted_iota(jnp.int32, (1, _CHUNK), 1)
    jcol = jax.lax.broadcasted_iota(jnp.int32, (_NOBJ, _CHUNK), 0)

    @pl.when(ph == 0)
    def _phase_a():
        px1 = pr_ref[0:1, :]
        py1 = pr_ref[1:2, :]
        px2 = pr_ref[2:3, :]
        py2 = pr_ref[3:4, :]
        bx1 = ox1_ref[0]
        by1 = oy1_ref[0]
        bx2 = ox2_ref[0]
        by2 = oy2_ref[0]
        iw = jnp.maximum(jnp.minimum(bx2, px2) - jnp.maximum(bx1, px1), 0.0)
        ih = jnp.maximum(jnp.minimum(by2, py2) - jnp.maximum(by1, py1), 0.0)
        inter = iw * ih
        area_o = (bx2 - bx1) * (by2 - by1)
        area_p = (px2 - px1) * (py2 - py1)
        union = jnp.maximum(area_o + area_p - inter, 1e-10)
        iou = inter / union

        cm = jnp.max(iou, axis=0, keepdims=True)
        ca = jnp.min(jnp.where(iou == cm, jcol, _NOBJ), axis=0, keepdims=True)
        colmax[:, pl.ds(c * _CHUNK, _CHUNK)] = cm
        colarg[:, pl.ds(c * _CHUNK, _CHUNK)] = ca

        rm = jnp.max(iou, axis=1, keepdims=True)
        ri = jnp.min(jnp.where(iou == rm, glob, _NPAD), axis=1, keepdims=True)
        upd = rm > rval[...]
        rval[...] = jnp.where(upd, rm, rval[...])
        ridx[...] = jnp.where(upd, ri, ridx[...])

    @pl.when(ph == 1)
    def _phase_b():
        cm = colmax[:, pl.ds(c * _CHUNK, _CHUNK)]
        ca = colarg[:, pl.ds(c * _CHUNK, _CHUNK)]
        pfe = ridx[...]
        match = pfe == glob
        forced = jnp.max(jnp.where(match, 1, 0), axis=0, keepdims=True) > 0
        assigned = jnp.max(jnp.where(match, jcol, -1), axis=0, keepdims=True)

        caf = ca.astype(jnp.float32)
        asgf = assigned.astype(jnp.float32)
        aca0[...] += jnp.sum(jnp.where(match, caf, 0.0), axis=1, keepdims=True)
        aasg[...] += jnp.sum(jnp.where(match, asgf, 0.0), axis=1, keepdims=True)

        oh = (jcol == ca).astype(jnp.float32)
        bt = bt_ref[0]
        tl = jax.lax.dot_general(bt, oh, (((1,), (0,)), ((), ())),
                                 preferred_element_type=jnp.float32)
        pred = pred_ref[0]
        valid = glob < _NP
        loc_out[0, 0] += jnp.sum(jnp.where(valid, jnp.abs(pred - tl), 0.0))

        ap0[...] += jnp.sum(jnp.where(match, pred[0:1, :], 0.0), axis=1,
                            keepdims=True)
        ap1[...] += jnp.sum(jnp.where(match, pred[1:2, :], 0.0), axis=1,
                            keepdims=True)
        ap2[...] += jnp.sum(jnp.where(match, pred[2:3, :], 0.0), axis=1,
                            keepdims=True)
        ap3[...] += jnp.sum(jnp.where(match, pred[3:4, :], 0.0), axis=1,
                            keepdims=True)

        @pl.when(c == _NCH - 1)
        def _emit_meta():
            jrow = jax.lax.broadcasted_iota(jnp.int32, (_NOBJ, 1), 0)
            pp_out[...] = jnp.concatenate(
                [ap0[...], ap1[...], ap2[...], ap3[...]], axis=1)
            ca0t_out[...] = b * _NOBJ + aca0[...].astype(jnp.int32)
            win_out[...] = (aasg[...].astype(jnp.int32) == jrow).astype(
                jnp.float32)

        @pl.when(b == nb - 1)
        def _score():
            s0 = sc_ref[0:1, :]
            s1 = sc_ref[1:2, :]
            m = jnp.maximum(s0, s1)
            lse = m + jnp.log(jnp.exp(s0 - m) + jnp.exp(s1 - m))
            lbl = jnp.logical_or(forced, cm >= _THR)
            lp = jnp.where(lbl, s1, s0) - lse
            sco_out[0, 0] += jnp.sum(jnp.where(valid, lp, 0.0))


@functools.partial(
    pl.kernel,
    mesh=plsc.VectorSubcoreMesh(core_axis_name="c", subcore_axis_name="s"),
    out_type=jax.ShapeDtypeStruct((_NW * 16,), jnp.float32),
    compiler_params=pltpu.CompilerParams(needs_layout_passes=False),
    scratch_types=[
        pltpu.VMEM((_PPW, 1), jnp.int32),
        pltpu.VMEM((_PPW, 1), jnp.float32),
        pltpu.VMEM((_PPW, 4), jnp.float32),
        pltpu.VMEM((8 * _NOBJ * 4,), jnp.float32),
        pltpu.VMEM((16,), jnp.float32),
    ],
)
def _sc_corr(pp_hbm, ca0t_hbm, win_hbm, tab_hbm, out_hbm,
             ca0_v, win_v, rows_v, tab_v, acc_v):
    wid = lax.axis_index("s") * 2 + lax.axis_index("c")
    b = wid >> 2
    q = (wid & 3) * _PPW
    woff = pl.multiple_of(wid * _PPW, 8)
    pltpu.sync_copy(pp_hbm.at[pl.ds(woff, _PPW)], rows_v)
    pltpu.sync_copy(ca0t_hbm.at[pl.ds(woff, _PPW)], ca0_v)
    pltpu.sync_copy(win_hbm.at[pl.ds(woff, _PPW)], win_v)
    pltpu.sync_copy(tab_hbm, tab_v)
    lane = lax.iota(jnp.int32, 16)
    lz = lane * 0
    asg_t = (b * _NOBJ + q + lane) * 4
    ca0_i = plsc.load_gather(ca0_v, [lane, lz])
    win = plsc.load_gather(win_v, [lane, lz])
    corr = jnp.zeros((16,), jnp.float32)
    for cc in range(4):
        p = plsc.load_gather(rows_v, [lane, lz + cc])
        ta = plsc.load_gather(tab_v, [asg_t + cc])
        tc0 = plsc.load_gather(tab_v, [ca0_i * 4 + cc])
        corr = corr + win * (jnp.abs(p - ta) - jnp.abs(p - tc0))
    acc_v[...] = corr
    pltpu.sync_copy(acc_v, out_hbm.at[pl.ds(pl.multiple_of(wid * 16, 8), 16)])


def kernel(predicted_boxes, predicted_scores, boxes, prior_boxes):
    bsz = predicted_boxes.shape[0]
    pad = _NPAD - _NP
    prT = jnp.pad(prior_boxes, ((0, pad), (0, 0))).T
    ox1 = boxes[..., 0:1]
    oy1 = boxes[..., 1:2]
    ox2 = boxes[..., 2:3]
    oy2 = boxes[..., 3:4]
    bT = jnp.swapaxes(boxes, 1, 2)
    predT = jnp.swapaxes(
        jnp.pad(predicted_boxes, ((0, 0), (0, pad), (0, 0))), 1, 2)
    scT = jnp.pad(predicted_scores, ((0, pad), (0, 0))).T

    pp, ca0t, win, loc_sum, sco_sum = pl.pallas_call(
        _tc_body,
        grid=(bsz, 2, _NCH),
        in_specs=[
            pl.BlockSpec((4, _CHUNK), lambda b, ph, c: (0, c)),
            pl.BlockSpec((1, _NOBJ, 1), lambda b, ph, c: (b, 0, 0)),
            pl.BlockSpec((1, _NOBJ, 1), lambda b, ph, c: (b, 0, 0)),
            pl.BlockSpec((1, _NOBJ, 1), lambda b, ph, c: (b, 0, 0)),
            pl.BlockSpec((1, _NOBJ, 1), lambda b, ph, c: (b, 0, 0)),
            pl.BlockSpec((1, 4, _NOBJ), lambda b, ph, c: (b, 0, 0)),
            pl.BlockSpec((1, 4, _CHUNK), lambda b, ph, c: (b, 0, c)),
            pl.BlockSpec((2, _CHUNK), lambda b, ph, c: (0, c)),
        ],
        out_specs=[
            pl.BlockSpec((_NOBJ, 4), lambda b, ph, c: (b, 0)),
            pl.BlockSpec((_NOBJ, 1), lambda b, ph, c: (b, 0)),
            pl.BlockSpec((_NOBJ, 1), lambda b, ph, c: (b, 0)),
            pl.BlockSpec((1, 1), lambda b, ph, c: (0, 0),
                         memory_space=pltpu.SMEM),
            pl.BlockSpec((1, 1), lambda b, ph, c: (0, 0),
                         memory_space=pltpu.SMEM),
        ],
        out_shape=[
            jax.ShapeDtypeStruct((bsz * _NOBJ, 4), jnp.float32),
            jax.ShapeDtypeStruct((bsz * _NOBJ, 1), jnp.int32),
            jax.ShapeDtypeStruct((bsz * _NOBJ, 1), jnp.float32),
            jax.ShapeDtypeStruct((1, 1), jnp.float32),
            jax.ShapeDtypeStruct((1, 1), jnp.float32),
        ],
        scratch_shapes=[
            pltpu.VMEM((1, _NPAD), jnp.float32),
            pltpu.VMEM((1, _NPAD), jnp.int32),
            pltpu.VMEM((_NOBJ, 1), jnp.float32),
            pltpu.VMEM((_NOBJ, 1), jnp.int32),
            pltpu.VMEM((_NOBJ, 1), jnp.float32),
            pltpu.VMEM((_NOBJ, 1), jnp.float32),
            pltpu.VMEM((_NOBJ, 1), jnp.float32),
            pltpu.VMEM((_NOBJ, 1), jnp.float32),
            pltpu.VMEM((_NOBJ, 1), jnp.float32),
            pltpu.VMEM((_NOBJ, 1), jnp.float32),
        ],
    )(prT, ox1, oy1, ox2, oy2, bT, predT, scT)

    tab = boxes.reshape(bsz * _NOBJ * 4)
    corr_parts = _sc_corr(pp, ca0t, win, tab)

    loc_loss = (loc_sum[0, 0] + jnp.sum(corr_parts)) / (bsz * _NP * 4)
    score_loss = -sco_sum[0, 0] / _NP
    return score_loss + loc_loss


# speedup vs baseline: 2.2076x; 1.0085x over previous
import functools

import jax
import jax.numpy as jnp
from jax import lax
from jax.experimental import pallas as pl
from jax.experimental.pallas import tpu as pltpu
from jax.experimental.pallas import tpu_sc as plsc

_NP = 20000
_NPAD = 20480
_CHUNK = 20480
_NCH = _NPAD // _CHUNK
_NOBJ = 64
_THR = 0.6
_NW = 32
_PPW = 16


def _tc_body(pr_ref, ox1_ref, oy1_ref, ox2_ref, oy2_ref, bt_ref, pred_ref,
             sc_ref, pp_out, ca0t_out, win_out, loc_out, sco_out,
             colmax, colarg, rval, ridx, aca0, aasg, ap0, ap1, ap2, ap3):
    b = pl.program_id(0)
    ph = pl.program_id(1)
    c = pl.program_id(2)
    nb = pl.num_programs(0)

    @pl.when(jnp.logical_and(jnp.logical_and(b == 0, ph == 0), c == 0))
    def _init():
        loc_out[0, 0] = 0.0
        sco_out[0, 0] = 0.0

    @pl.when(jnp.logical_and(ph == 0, c == 0))
    def _reset():
        rval[...] = jnp.full_like(rval[...], -1.0)
        ridx[...] = jnp.zeros_like(ridx[...])

    @pl.when(jnp.logical_and(ph == 1, c == 0))
    def _reset_b():
        aca0[...] = jnp.zeros_like(aca0[...])
        aasg[...] = jnp.zeros_like(aasg[...])
        ap0[...] = jnp.zeros_like(ap0[...])
        ap1[...] = jnp.zeros_like(ap1[...])
        ap2[...] = jnp.zeros_like(ap2[...])
        ap3[...] = jnp.zeros_like(ap3[...])

    glob = c * _CHUNK + jax.lax.broadcasted_iota(jnp.int32, (1, _CHUNK), 1)
    jcol = jax.lax.broadcasted_iota(jnp.int32, (_NOBJ, _CHUNK), 0)

    @pl.when(ph == 0)
    def _phase_a():
        px1 = pr_ref[0:1, :]
        py1 = pr_ref[1:2, :]
        px2 = pr_ref[2:3, :]
        py2 = pr_ref[3:4, :]
        bx1 = ox1_ref[0]
        by1 = oy1_ref[0]
        bx2 = ox2_ref[0]
        by2 = oy2_ref[0]
        iw = jnp.maximum(jnp.minimum(bx2, px2) - jnp.maximum(bx1, px1), 0.0)
        ih = jnp.maximum(jnp.minimum(by2, py2) - jnp.maximum(by1, py1), 0.0)
        inter = iw * ih
        area_o = (bx2 - bx1) * (by2 - by1)
        area_p = (px2 - px1) * (py2 - py1)
        union = jnp.maximum(area_o + area_p - inter, 1e-10)
        iou = inter / union

        cm = jnp.max(iou, axis=0, keepdims=True)
        ca = jnp.min(jnp.where(iou == cm, jcol, _NOBJ), axis=0, keepdims=True)
        colmax[:, pl.ds(c * _CHUNK, _CHUNK)] = cm
        colarg[:, pl.ds(c * _CHUNK, _CHUNK)] = ca

        rm = jnp.max(iou, axis=1, keepdims=True)
        ri = jnp.min(jnp.where(iou == rm, glob, _NPAD), axis=1, keepdims=True)
        upd = rm > rval[...]
        rval[...] = jnp.where(upd, rm, rval[...])
        ridx[...] = jnp.where(upd, ri, ridx[...])

    @pl.when(ph == 1)
    def _phase_b():
        cm = colmax[:, pl.ds(c * _CHUNK, _CHUNK)]
        ca = colarg[:, pl.ds(c * _CHUNK, _CHUNK)]
        pfe = ridx[...]
        match = pfe == glob
        forced = jnp.max(jnp.where(match, 1, 0), axis=0, keepdims=True) > 0
        assigned = jnp.max(jnp.where(match, jcol, -1), axis=0, keepdims=True)

        caf = ca.astype(jnp.float32)
        asgf = assigned.astype(jnp.float32)
        aca0[...] += jnp.sum(jnp.where(match, caf, 0.0), axis=1, keepdims=True)
        aasg[...] += jnp.sum(jnp.where(match, asgf, 0.0), axis=1, keepdims=True)

        oh = (jcol == ca).astype(jnp.float32)
        bt = bt_ref[0]
        tl = jax.lax.dot_general(bt, oh, (((1,), (0,)), ((), ())),
                                 preferred_element_type=jnp.float32)
        pred = pred_ref[0]
        valid = glob < _NP
        loc_out[0, 0] += jnp.sum(jnp.where(valid, jnp.abs(pred - tl), 0.0))

        ap0[...] += jnp.sum(jnp.where(match, pred[0:1, :], 0.0), axis=1,
                            keepdims=True)
        ap1[...] += jnp.sum(jnp.where(match, pred[1:2, :], 0.0), axis=1,
                            keepdims=True)
        ap2[...] += jnp.sum(jnp.where(match, pred[2:3, :], 0.0), axis=1,
                            keepdims=True)
        ap3[...] += jnp.sum(jnp.where(match, pred[3:4, :], 0.0), axis=1,
                            keepdims=True)

        @pl.when(c == _NCH - 1)
        def _emit_meta():
            jrow = jax.lax.broadcasted_iota(jnp.int32, (_NOBJ, 1), 0)
            pp_out[...] = jnp.concatenate(
                [ap0[...], ap1[...], ap2[...], ap3[...]], axis=1)
            ca0t_out[...] = b * _NOBJ + aca0[...].astype(jnp.int32)
            win_out[...] = (aasg[...].astype(jnp.int32) == jrow).astype(
                jnp.float32)

        @pl.when(b == nb - 1)
        def _score():
            s0 = sc_ref[0:1, :]
            s1 = sc_ref[1:2, :]
            m = jnp.maximum(s0, s1)
            lse = m + jnp.log(jnp.exp(s0 - m) + jnp.exp(s1 - m))
            lbl = jnp.logical_or(forced, cm >= _THR)
            lp = jnp.where(lbl, s1, s0) - lse
            sco_out[0, 0] += jnp.sum(jnp.where(valid, lp, 0.0))


@functools.partial(
    pl.kernel,
    mesh=plsc.VectorSubcoreMesh(core_axis_name="c", subcore_axis_name="s"),
    out_type=jax.ShapeDtypeStruct((_NW * 16,), jnp.float32),
    compiler_params=pltpu.CompilerParams(needs_layout_passes=False),
    scratch_types=[
        pltpu.VMEM((_PPW, 1), jnp.int32),
        pltpu.VMEM((_PPW, 1), jnp.float32),
        pltpu.VMEM((_PPW, 4), jnp.float32),
        pltpu.VMEM((8 * _NOBJ * 4,), jnp.float32),
        pltpu.VMEM((16,), jnp.float32),
    ],
)
def _sc_corr(pp_hbm, ca0t_hbm, win_hbm, tab_hbm, out_hbm,
             ca0_v, win_v, rows_v, tab_v, acc_v):
    wid = lax.axis_index("s") * 2 + lax.axis_index("c")
    b = wid >> 2
    q = (wid & 3) * _PPW
    woff = pl.multiple_of(wid * _PPW, 8)
    pltpu.sync_copy(pp_hbm.at[pl.ds(woff, _PPW)], rows_v)
    pltpu.sync_copy(ca0t_hbm.at[pl.ds(woff, _PPW)], ca0_v)
    pltpu.sync_copy(win_hbm.at[pl.ds(woff, _PPW)], win_v)
    pltpu.sync_copy(tab_hbm, tab_v)
    lane = lax.iota(jnp.int32, 16)
    lz = lane * 0
    asg_t = (b * _NOBJ + q + lane) * 4
    ca0_i = plsc.load_gather(ca0_v, [lane, lz])
    win = plsc.load_gather(win_v, [lane, lz])
    corr = jnp.zeros((16,), jnp.float32)
    for cc in range(4):
        p = plsc.load_gather(rows_v, [lane, lz + cc])
        ta = plsc.load_gather(tab_v, [asg_t + cc])
        tc0 = plsc.load_gather(tab_v, [ca0_i * 4 + cc])
        corr = corr + win * (jnp.abs(p - ta) - jnp.abs(p - tc0))
    acc_v[...] = corr
    pltpu.sync_copy(acc_v, out_hbm.at[pl.ds(pl.multiple_of(wid * 16, 8), 16)])


def kernel(predicted_boxes, predicted_scores, boxes, prior_boxes):
    bsz = predicted_boxes.shape[0]
    pad = _NPAD - _NP
    prT = jnp.pad(prior_boxes, ((0, pad), (0, 0))).T
    ox1 = boxes[..., 0:1]
    oy1 = boxes[..., 1:2]
    ox2 = boxes[..., 2:3]
    oy2 = boxes[..., 3:4]
    bT = jnp.swapaxes(boxes, 1, 2)
    predT = jnp.swapaxes(
        jnp.pad(predicted_boxes, ((0, 0), (0, pad), (0, 0))), 1, 2)
    scT = jnp.pad(predicted_scores, ((0, pad), (0, 0))).T

    pp, ca0t, win, loc_sum, sco_sum = pl.pallas_call(
        _tc_body,
        grid=(bsz, 2, _NCH),
        in_specs=[
            pl.BlockSpec((4, _CHUNK), lambda b, ph, c: (0, c)),
            pl.BlockSpec((1, _NOBJ, 1), lambda b, ph, c: (b, 0, 0)),
            pl.BlockSpec((1, _NOBJ, 1), lambda b, ph, c: (b, 0, 0)),
            pl.BlockSpec((1, _NOBJ, 1), lambda b, ph, c: (b, 0, 0)),
            pl.BlockSpec((1, _NOBJ, 1), lambda b, ph, c: (b, 0, 0)),
            pl.BlockSpec((1, 4, _NOBJ), lambda b, ph, c: (b, 0, 0)),
            pl.BlockSpec((1, 4, _CHUNK), lambda b, ph, c: (b, 0, c)),
            pl.BlockSpec((2, _CHUNK), lambda b, ph, c: (0, c)),
        ],
        out_specs=[
            pl.BlockSpec((_NOBJ, 4), lambda b, ph, c: (b, 0)),
            pl.BlockSpec((_NOBJ, 1), lambda b, ph, c: (b, 0)),
            pl.BlockSpec((_NOBJ, 1), lambda b, ph, c: (b, 0)),
            pl.BlockSpec((1, 1), lambda b, ph, c: (0, 0),
                         memory_space=pltpu.SMEM),
            pl.BlockSpec((1, 1), lambda b, ph, c: (0, 0),
                         memory_space=pltpu.SMEM),
        ],
        out_shape=[
            jax.ShapeDtypeStruct((bsz * _NOBJ, 4), jnp.float32),
            jax.ShapeDtypeStruct((bsz * _NOBJ, 1), jnp.int32),
            jax.ShapeDtypeStruct((bsz * _NOBJ, 1), jnp.float32),
            jax.ShapeDtypeStruct((1, 1), jnp.float32),
            jax.ShapeDtypeStruct((1, 1), jnp.float32),
        ],
        scratch_shapes=[
            pltpu.VMEM((1, _NPAD), jnp.float32),
            pltpu.VMEM((1, _NPAD), jnp.int32),
            pltpu.VMEM((_NOBJ, 1), jnp.float32),
            pltpu.VMEM((_NOBJ, 1), jnp.int32),
            pltpu.VMEM((_NOBJ, 1), jnp.float32),
            pltpu.VMEM((_NOBJ, 1), jnp.float32),
            pltpu.VMEM((_NOBJ, 1), jnp.float32),
            pltpu.VMEM((_NOBJ, 1), jnp.float32),
            pltpu.VMEM((_NOBJ, 1), jnp.float32),
            pltpu.VMEM((_NOBJ, 1), jnp.float32),
        ],
    )(prT, ox1, oy1, ox2, oy2, bT, predT, scT)

    tab = boxes.reshape(bsz * _NOBJ * 4)
    corr_parts = _sc_corr(pp, ca0t, win, tab)

    loc_loss = (loc_sum[0, 0] + jnp.sum(corr_parts)) / (bsz * _NP * 4)
    score_loss = -sco_sum[0, 0] / _NP
    return score_loss + loc_loss


# speedup vs baseline: 2.2188x; 1.0051x over previous
import functools

import jax
import jax.numpy as jnp
from jax import lax
from jax.experimental import pallas as pl
from jax.experimental.pallas import tpu as pltpu
from jax.experimental.pallas import tpu_sc as plsc

_NP = 20000
_NPAD = 20480
_NOBJ = 64
_THR = 0.6
_NW = 32
_PPW = 16


def _tc_body(pr_ref, ox1_ref, oy1_ref, ox2_ref, oy2_ref, bt_ref, pred_ref,
             sc_ref, pp_out, ca0t_out, win_out, loc_out, sco_out):
    b = pl.program_id(0)
    nb = pl.num_programs(0)

    @pl.when(b == 0)
    def _init():
        loc_out[0, 0] = 0.0
        sco_out[0, 0] = 0.0

    glob = jax.lax.broadcasted_iota(jnp.int32, (1, _NPAD), 1)
    jcol = jax.lax.broadcasted_iota(jnp.int32, (_NOBJ, _NPAD), 0)

    px1 = pr_ref[0:1, :]
    py1 = pr_ref[1:2, :]
    px2 = pr_ref[2:3, :]
    py2 = pr_ref[3:4, :]
    bx1 = ox1_ref[0]
    by1 = oy1_ref[0]
    bx2 = ox2_ref[0]
    by2 = oy2_ref[0]
    iw = jnp.maximum(jnp.minimum(bx2, px2) - jnp.maximum(bx1, px1), 0.0)
    ih = jnp.maximum(jnp.minimum(by2, py2) - jnp.maximum(by1, py1), 0.0)
    inter = iw * ih
    area_o = (bx2 - bx1) * (by2 - by1)
    area_p = (px2 - px1) * (py2 - py1)
    union = jnp.maximum(area_o + area_p - inter, 1e-10)
    iou = inter / union

    cm = jnp.max(iou, axis=0, keepdims=True)
    ca = jnp.min(jnp.where(iou == cm, jcol, _NOBJ), axis=0, keepdims=True)

    rm = jnp.max(iou, axis=1, keepdims=True)
    pfe = jnp.min(jnp.where(iou == rm, glob, _NPAD), axis=1, keepdims=True)

    match = pfe == glob
    forced = jnp.max(jnp.where(match, 1, 0), axis=0, keepdims=True) > 0
    assigned = jnp.max(jnp.where(match, jcol, -1), axis=0, keepdims=True)

    caf = ca.astype(jnp.float32)
    asgf = assigned.astype(jnp.float32)
    jrow = jax.lax.broadcasted_iota(jnp.int32, (_NOBJ, 1), 0)
    aca0 = jnp.sum(jnp.where(match, caf, 0.0), axis=1, keepdims=True)
    aasg = jnp.sum(jnp.where(match, asgf, 0.0), axis=1, keepdims=True)
    pred = pred_ref[0]
    ap = [jnp.sum(jnp.where(match, pred[i:i + 1, :], 0.0), axis=1,
                  keepdims=True) for i in range(4)]
    pp_out[...] = jnp.concatenate(ap, axis=1)
    ca0t_out[...] = b * _NOBJ + aca0.astype(jnp.int32)
    win_out[...] = (aasg.astype(jnp.int32) == jrow).astype(jnp.float32)

    oh = (jcol == ca).astype(jnp.float32)
    bt = bt_ref[0]
    tl = jax.lax.dot_general(bt, oh, (((1,), (0,)), ((), ())),
                             preferred_element_type=jnp.float32)
    valid = glob < _NP
    loc_out[0, 0] += jnp.sum(jnp.where(valid, jnp.abs(pred - tl), 0.0))

    @pl.when(b == nb - 1)
    def _score():
        s0 = sc_ref[0:1, :]
        s1 = sc_ref[1:2, :]
        m = jnp.maximum(s0, s1)
        lse = m + jnp.log(jnp.exp(s0 - m) + jnp.exp(s1 - m))
        lbl = jnp.logical_or(forced, cm >= _THR)
        lp = jnp.where(lbl, s1, s0) - lse
        sco_out[0, 0] += jnp.sum(jnp.where(valid, lp, 0.0))


@functools.partial(
    pl.kernel,
    mesh=plsc.VectorSubcoreMesh(core_axis_name="c", subcore_axis_name="s"),
    out_type=jax.ShapeDtypeStruct((_NW * 16,), jnp.float32),
    compiler_params=pltpu.CompilerParams(needs_layout_passes=False),
    scratch_types=[
        pltpu.VMEM((_PPW, 1), jnp.int32),
        pltpu.VMEM((_PPW, 1), jnp.float32),
        pltpu.VMEM((_PPW, 4), jnp.float32),
        pltpu.VMEM((8 * _NOBJ * 4,), jnp.float32),
        pltpu.VMEM((16,), jnp.float32),
    ],
)
def _sc_corr(pp_hbm, ca0t_hbm, win_hbm, tab_hbm, out_hbm,
             ca0_v, win_v, rows_v, tab_v, acc_v):
    wid = lax.axis_index("s") * 2 + lax.axis_index("c")
    b = wid >> 2
    q = (wid & 3) * _PPW
    woff = pl.multiple_of(wid * _PPW, 8)
    pltpu.sync_copy(pp_hbm.at[pl.ds(woff, _PPW)], rows_v)
    pltpu.sync_copy(ca0t_hbm.at[pl.ds(woff, _PPW)], ca0_v)
    pltpu.sync_copy(win_hbm.at[pl.ds(woff, _PPW)], win_v)
    pltpu.sync_copy(tab_hbm, tab_v)
    lane = lax.iota(jnp.int32, 16)
    lz = lane * 0
    asg_t = (b * _NOBJ + q + lane) * 4
    ca0_i = plsc.load_gather(ca0_v, [lane, lz])
    win = plsc.load_gather(win_v, [lane, lz])
    corr = jnp.zeros((16,), jnp.float32)
    for cc in range(4):
        p = plsc.load_gather(rows_v, [lane, lz + cc])
        ta = plsc.load_gather(tab_v, [asg_t + cc])
        tc0 = plsc.load_gather(tab_v, [ca0_i * 4 + cc])
        corr = corr + win * (jnp.abs(p - ta) - jnp.abs(p - tc0))
    acc_v[...] = corr
    pltpu.sync_copy(acc_v, out_hbm.at[pl.ds(pl.multiple_of(wid * 16, 8), 16)])


def kernel(predicted_boxes, predicted_scores, boxes, prior_boxes):
    bsz = predicted_boxes.shape[0]
    pad = _NPAD - _NP
    prT = jnp.pad(prior_boxes, ((0, pad), (0, 0))).T
    ox1 = boxes[..., 0:1]
    oy1 = boxes[..., 1:2]
    ox2 = boxes[..., 2:3]
    oy2 = boxes[..., 3:4]
    bT = jnp.swapaxes(boxes, 1, 2)
    predT = jnp.swapaxes(
        jnp.pad(predicted_boxes, ((0, 0), (0, pad), (0, 0))), 1, 2)
    scT = jnp.pad(predicted_scores, ((0, pad), (0, 0))).T

    pp, ca0t, win, loc_sum, sco_sum = pl.pallas_call(
        _tc_body,
        grid=(bsz,),
        in_specs=[
            pl.BlockSpec((4, _NPAD), lambda b: (0, 0)),
            pl.BlockSpec((1, _NOBJ, 1), lambda b: (b, 0, 0)),
            pl.BlockSpec((1, _NOBJ, 1), lambda b: (b, 0, 0)),
            pl.BlockSpec((1, _NOBJ, 1), lambda b: (b, 0, 0)),
            pl.BlockSpec((1, _NOBJ, 1), lambda b: (b, 0, 0)),
            pl.BlockSpec((1, 4, _NOBJ), lambda b: (b, 0, 0)),
            pl.BlockSpec((1, 4, _NPAD), lambda b: (b, 0, 0)),
            pl.BlockSpec((2, _NPAD), lambda b: (0, 0)),
        ],
        out_specs=[
            pl.BlockSpec((_NOBJ, 4), lambda b: (b, 0)),
            pl.BlockSpec((_NOBJ, 1), lambda b: (b, 0)),
            pl.BlockSpec((_NOBJ, 1), lambda b: (b, 0)),
            pl.BlockSpec((1, 1), lambda b: (0, 0), memory_space=pltpu.SMEM),
            pl.BlockSpec((1, 1), lambda b: (0, 0), memory_space=pltpu.SMEM),
        ],
        out_shape=[
            jax.ShapeDtypeStruct((bsz * _NOBJ, 4), jnp.float32),
            jax.ShapeDtypeStruct((bsz * _NOBJ, 1), jnp.int32),
            jax.ShapeDtypeStruct((bsz * _NOBJ, 1), jnp.float32),
            jax.ShapeDtypeStruct((1, 1), jnp.float32),
            jax.ShapeDtypeStruct((1, 1), jnp.float32),
        ],
    )(prT, ox1, oy1, ox2, oy2, bT, predT, scT)

    tab = boxes.reshape(bsz * _NOBJ * 4)
    corr_parts = _sc_corr(pp, ca0t, win, tab)

    loc_loss = (loc_sum[0, 0] + jnp.sum(corr_parts)) / (bsz * _NP * 4)
    score_loss = -sco_sum[0, 0] / _NP
    return score_loss + loc_loss
